# baseline (speedup 1.0000x reference)
import jax
import jax.numpy as jnp
from jax.experimental import pallas as pl

NOUT = 32
EHID = 128


def _we_body(ef_ref, we1_ref, be1_ref, we2_ref, be2_ref, out_ref):
    g = jnp.maximum(ef_ref[...] @ we1_ref[...] + be1_ref[...], 0.0)
    out_ref[...] = g @ we2_ref[...] + be2_ref[...]


def kernel(node_feats, edge_feats, edge_index, W_proj, b_proj, W_e1, b_e1,
           W_e2, b_e2, b_conv, W_ih, b_ih, W_hh, b_hh, W_pred, b_pred):
    E = edge_feats.shape[0]
    n_nodes = node_feats.shape[0]
    BE = 2000
    we = pl.pallas_call(
        _we_body,
        grid=(E // BE,),
        in_specs=[
            pl.BlockSpec((BE, 16), lambda i: (i, 0)),
            pl.BlockSpec((16, EHID), lambda i: (0, 0)),
            pl.BlockSpec((1, EHID), lambda i: (0, 0)),
            pl.BlockSpec((EHID, NOUT * NOUT), lambda i: (0, 0)),
            pl.BlockSpec((1, NOUT * NOUT), lambda i: (0, 0)),
        ],
        out_specs=pl.BlockSpec((BE, NOUT * NOUT), lambda i: (i, 0)),
        out_shape=jax.ShapeDtypeStruct((E, NOUT * NOUT), jnp.float32),
    )(edge_feats, W_e1, b_e1.reshape(1, -1), W_e2, b_e2.reshape(1, -1))
    We = we.reshape(-1, NOUT, NOUT)

    src = edge_index[0]
    dst = edge_index[1]
    h = jax.nn.relu(node_feats @ W_proj + b_proj)
    hidden = h
    for _ in range(6):
        m = jnp.einsum('ei,eio->eo', h[src], We)
        agg = jax.ops.segment_sum(m, dst, num_segments=n_nodes)
        x = jax.nn.relu(agg + b_conv)
        gi = x @ W_ih + b_ih
        gh = hidden @ W_hh + b_hh
        i_r, i_z, i_n = jnp.split(gi, 3, axis=-1)
        h_r, h_z, h_n = jnp.split(gh, 3, axis=-1)
        r = jax.nn.sigmoid(i_r + h_r)
        z = jax.nn.sigmoid(i_z + h_z)
        n = jnp.tanh(i_n + r * h_n)
        hidden = (1.0 - z) * n + z * hidden
        h = hidden
    return jnp.concatenate([h[src], h[dst]], axis=1) @ W_pred + b_pred


# speedup vs baseline: 1.9404x; 1.9404x over previous
import functools

import jax
import jax.numpy as jnp
from jax import lax
from jax.experimental import pallas as pl
from jax.experimental.pallas import tpu as pltpu
from jax.experimental.pallas import tpu_sc as plsc

N_NODES = 10000
D_NODE = 128
D_EDGE = 16
NOUT = 32
EHID = 128
STEPS = 6

NC, NS = 2, 16
NW = NC * NS
NP = 10240
NPW = NP // NW
EP = 163840
EPW = EP // NW
CE = 512
BE = 2048
BN = 2048

_MESH = dict(core_axis_name="c", subcore_axis_name="s")


def _wid():
    return lax.axis_index("s") * NC + lax.axis_index("c")


def _permute_rows_body(ef_hbm, perm_hbm, out_hbm, idx_v, rows_v, sem):
    base_w = _wid() * EPW

    def chunk(c, _):
        base = base_w + c * CE
        pltpu.sync_copy(perm_hbm.at[pl.ds(base, CE)], idx_v)
        pltpu.async_copy(ef_hbm.at[idx_v], rows_v, sem).wait()
        pltpu.sync_copy(rows_v, out_hbm.at[pl.ds(base, CE)])
        return 0

    lax.fori_loop(0, EPW // CE, chunk, 0, unroll=False)


def _gather_rows_body(h_hbm, src_hbm, out_hbm, idx_v, rows_v, sem):
    base_w = _wid() * EPW

    def chunk(c, _):
        base = base_w + c * CE
        pltpu.sync_copy(src_hbm.at[pl.ds(base, CE)], idx_v)
        pltpu.async_copy(h_hbm.at[idx_v], rows_v, sem).wait()
        pltpu.sync_copy(rows_v, out_hbm.at[pl.ds(base, CE)])
        return 0

    lax.fori_loop(0, EPW // CE, chunk, 0, unroll=False)


def _pick(offs_v, target):
    ii = lax.iota(jnp.int32, 16)
    r = jnp.full((), -1, jnp.int32)
    for b in range(3):
        vb = offs_v[pl.ds(b * 16, 16)]
        r = jnp.maximum(
            r, jnp.max(jnp.where(ii + (b * 16) == target, vb, -1), axis=0))
    return r


def _scatter_body(mT_hbm, dst_hbm, offs_hbm, out_hbm, offs_v, dstC_v, mloc_v,
                  aggL_v, sem):
    w = _wid()
    pltpu.sync_copy(offs_hbm, offs_v)
    lo_node = w * NPW
    e_lo_raw = _pick(offs_v, w)
    e_hi = _pick(offs_v, w + 1)
    e_lo = (e_lo_raw // 16) * 16

    def zero(i, _):
        aggL_v[pl.ds(i * 16, 16)] = jnp.zeros((16,), jnp.float32)
        return 0

    lax.fori_loop(0, NPW * NOUT // 16, zero, 0, unroll=False)

    nch = (e_hi - e_lo + CE - 1) // CE

    def chunk(c, _):
        cstart = e_lo + c * CE
        base = pl.multiple_of(jnp.minimum(cstart, EP - CE), 16)
        pltpu.sync_copy(dst_hbm.at[pl.ds(base, CE)], dstC_v)
        pltpu.sync_copy(mT_hbm.at[:, pl.ds(base, CE)], mloc_v)

        def grp(g, _):
            gidx = base + g * 16 + lax.iota(jnp.int32, 16)
            dvec = dstC_v[pl.ds(g * 16, 16)]
            mask = ((dvec >= lo_node) & (dvec < lo_node + NPW)
                    & (gidx >= cstart))
            addr0 = (dvec - lo_node) * NOUT
            for o in range(NOUT):
                vals = mloc_v[o, pl.ds(g * 16, 16)]
                plsc.addupdate_scatter(aggL_v, [addr0 + o], vals, mask=mask)
            return 0

        lax.fori_loop(0, CE // 16, grp, 0, unroll=False)
        return 0

    lax.fori_loop(0, nch, chunk, 0, unroll=False)
    pltpu.sync_copy(aggL_v, out_hbm.at[pl.ds(w * NPW * NOUT, NPW * NOUT)])


def _score_body(uv_hbm, src_hbm, dst_hbm, out_hbm, uv_v, srcC_v, dstC_v,
                s_v, sem):
    base_w = _wid() * EPW
    pltpu.sync_copy(uv_hbm, uv_v)

    def chunk(c, _):
        base = base_w + c * CE
        pltpu.sync_copy(src_hbm.at[pl.ds(base, CE)], srcC_v)
        pltpu.sync_copy(dst_hbm.at[pl.ds(base, CE)], dstC_v)

        def grp(g, _):
            sl = pl.ds(g * 16, 16)
            zz = jnp.zeros((16,), jnp.int32)
            u16 = plsc.load_gather(uv_v, [srcC_v[sl], zz])
            v16 = plsc.load_gather(uv_v, [dstC_v[sl], zz + 1])
            s_v[sl] = u16 + v16
            return 0

        lax.fori_loop(0, CE // 16, grp, 0, unroll=False)
        pltpu.sync_copy(s_v, out_hbm.at[pl.ds(base, CE)])
        return 0

    lax.fori_loop(0, EPW // CE, chunk, 0, unroll=False)


def _sc_kernel(body, out_type, scratch_types):
    return pl.kernel(body, out_type=out_type,
                     mesh=plsc.VectorSubcoreMesh(**_MESH),
                     scratch_types=scratch_types,
                     compiler_params=pltpu.CompilerParams(
                         use_tc_tiling_on_sc=False,
                         needs_layout_passes=False))


def _mm(a, b):
    return jnp.matmul(a, b)


def _proj_body(nf_ref, w_ref, b_ref, out_ref):
    out_ref[...] = jnp.maximum(_mm(nf_ref[...], w_ref[...]) + b_ref[...], 0.0)


def _wesoa_body(ef_ref, w1_ref, b1_ref, w2_ref, b2_ref, out_ref):
    g = jnp.maximum(_mm(ef_ref[...], w1_ref[...]) + b1_ref[...], 0.0)
    out_ref[...] = lax.dot_general(
        w2_ref[...], g, (((0,), (1,)), ((), ())),
        preferred_element_type=jnp.float32) + b2_ref[...]


def _einsum_body(we_ref, hs_ref, out_ref):
    hT = jnp.transpose(hs_ref[...])
    terms = [we_ref[pl.ds(i * NOUT, NOUT), :] * hT[i:i + 1, :]
             for i in range(NOUT)]
    while len(terms) > 1:
        terms = [terms[j] + terms[j + 1] for j in range(0, len(terms), 2)]
    out_ref[...] = terms[0]


def _gru_body(agg_ref, hid_ref, bconv_ref, wih_ref, bih_ref, whh_ref,
              bhh_ref, out_ref):
    x = jnp.maximum(agg_ref[...] + bconv_ref[...], 0.0)
    hid = hid_ref[...]
    gi = _mm(x, wih_ref[...]) + bih_ref[...]
    gh = _mm(hid, whh_ref[...]) + bhh_ref[...]
    r = jax.nn.sigmoid(gi[:, :NOUT] + gh[:, :NOUT])
    z = jax.nn.sigmoid(gi[:, NOUT:2 * NOUT] + gh[:, NOUT:2 * NOUT])
    n = jnp.tanh(gi[:, 2 * NOUT:] + r * gh[:, 2 * NOUT:])
    out_ref[...] = (1.0 - z) * n + z * hid


def _uv_body(h_ref, wab_ref, badd_ref, out_ref):
    out_ref[...] = _mm(h_ref[...], wab_ref[...]) + badd_ref[...]


def _rep(shape, idx):
    return pl.BlockSpec(shape, idx)


def kernel(node_feats, edge_feats, edge_index, W_proj, b_proj, W_e1, b_e1,
           W_e2, b_e2, b_conv, W_ih, b_ih, W_hh, b_hh, W_pred, b_pred):
    f32 = jnp.float32
    src = edge_index[0]
    dst = edge_index[1]
    E = src.shape[0]

    perm = jnp.argsort(dst)
    dst_s = jnp.concatenate([dst[perm],
                             jnp.full((EP - E,), NP - 1, jnp.int32)])
    src_s = jnp.concatenate([src[perm], jnp.zeros((EP - E,), jnp.int32)])
    perm_p = jnp.concatenate([perm.astype(jnp.int32),
                              jnp.zeros((EP - E,), jnp.int32)])
    offs = jnp.searchsorted(dst_s, jnp.arange(33, dtype=jnp.int32) * NPW,
                            side="left").astype(jnp.int32)
    offs = jnp.concatenate([offs, jnp.zeros((31,), jnp.int32)])
    src_n = jnp.concatenate([src, jnp.zeros((EP - E,), jnp.int32)])
    dst_n = jnp.concatenate([dst, jnp.zeros((EP - E,), jnp.int32)])
    nf_p = jnp.pad(node_feats, ((0, NP - N_NODES), (0, 0)))

    ef_s = _sc_kernel(
        _permute_rows_body, jax.ShapeDtypeStruct((EP, D_EDGE), f32),
        [pltpu.VMEM((CE,), jnp.int32), pltpu.VMEM((CE, D_EDGE), f32),
         pltpu.SemaphoreType.DMA],
    )(edge_feats, perm_p)

    h0 = pl.pallas_call(
        _proj_body, grid=(NP // BN,),
        in_specs=[_rep((BN, D_NODE), lambda i: (i, 0)),
                  _rep((D_NODE, NOUT), lambda i: (0, 0)),
                  _rep((1, NOUT), lambda i: (0, 0))],
        out_specs=_rep((BN, NOUT), lambda i: (i, 0)),
        out_shape=jax.ShapeDtypeStruct((NP, NOUT), f32),
    )(nf_p, W_proj, b_proj.reshape(1, -1))

    we_soa = pl.pallas_call(
        _wesoa_body, grid=(EP // BE,),
        in_specs=[_rep((BE, D_EDGE), lambda i: (i, 0)),
                  _rep((D_EDGE, EHID), lambda i: (0, 0)),
                  _rep((1, EHID), lambda i: (0, 0)),
                  _rep((EHID, NOUT * NOUT), lambda i: (0, 0)),
                  _rep((NOUT * NOUT, 1), lambda i: (0, 0))],
        out_specs=_rep((NOUT * NOUT, BE), lambda i: (0, i)),
        out_shape=jax.ShapeDtypeStruct((NOUT * NOUT, EP), f32),
    )(ef_s, W_e1, b_e1.reshape(1, -1), W_e2, b_e2.reshape(-1, 1))

    gather_rows = _sc_kernel(
        _gather_rows_body, jax.ShapeDtypeStruct((EP, NOUT), f32),
        [pltpu.VMEM((CE,), jnp.int32), pltpu.VMEM((CE, NOUT), f32),
         pltpu.SemaphoreType.DMA],
    )
    scatter_agg = _sc_kernel(
        _scatter_body, jax.ShapeDtypeStruct((NP * NOUT,), f32),
        [pltpu.VMEM((64,), jnp.int32), pltpu.VMEM((CE,), jnp.int32),
         pltpu.VMEM((NOUT, CE), f32), pltpu.VMEM((NPW * NOUT,), f32),
         pltpu.SemaphoreType.DMA],
    )
    einsum_mT = pl.pallas_call(
        _einsum_body, grid=(EP // BE,),
        in_specs=[_rep((NOUT * NOUT, BE), lambda i: (0, i)),
                  _rep((BE, NOUT), lambda i: (i, 0))],
        out_specs=_rep((NOUT, BE), lambda i: (0, i)),
        out_shape=jax.ShapeDtypeStruct((NOUT, EP), f32),
    )
    gru = pl.pallas_call(
        _gru_body, grid=(NP // BN,),
        in_specs=[_rep((BN, NOUT), lambda i: (i, 0)),
                  _rep((BN, NOUT), lambda i: (i, 0)),
                  _rep((1, NOUT), lambda i: (0, 0)),
                  _rep((NOUT, 3 * NOUT), lambda i: (0, 0)),
                  _rep((1, 3 * NOUT), lambda i: (0, 0)),
                  _rep((NOUT, 3 * NOUT), lambda i: (0, 0)),
                  _rep((1, 3 * NOUT), lambda i: (0, 0))],
        out_specs=_rep((BN, NOUT), lambda i: (i, 0)),
        out_shape=jax.ShapeDtypeStruct((NP, NOUT), f32),
    )

    hid = h0
    for _ in range(STEPS):
        hs = gather_rows(hid, src_s)
        mT = einsum_mT(we_soa, hs)
        agg = scatter_agg(mT, dst_s, offs).reshape(NP, NOUT)
        hid = gru(agg, hid, b_conv.reshape(1, -1), W_ih,
                  b_ih.reshape(1, -1), W_hh, b_hh.reshape(1, -1))

    wab = jnp.concatenate([W_pred[:NOUT], W_pred[NOUT:]], axis=1)
    badd = jnp.stack([b_pred[0], jnp.zeros((), f32)]).reshape(1, 2)
    uv = pl.pallas_call(
        _uv_body, grid=(NP // BN,),
        in_specs=[_rep((BN, NOUT), lambda i: (i, 0)),
                  _rep((NOUT, 2), lambda i: (0, 0)),
                  _rep((1, 2), lambda i: (0, 0))],
        out_specs=_rep((BN, 2), lambda i: (i, 0)),
        out_shape=jax.ShapeDtypeStruct((NP, 2), f32),
    )(hid, wab, badd)

    score = _sc_kernel(
        _score_body, jax.ShapeDtypeStruct((EP,), f32),
        [pltpu.VMEM((NP, 2), f32), pltpu.VMEM((CE,), jnp.int32),
         pltpu.VMEM((CE,), jnp.int32), pltpu.VMEM((CE,), f32),
         pltpu.SemaphoreType.DMA],
    )(uv, src_n, dst_n)

    return score[:E].reshape(E, 1)


# speedup vs baseline: 1.9524x; 1.0062x over previous
import functools

import jax
import jax.numpy as jnp
from jax import lax
from jax.experimental import pallas as pl
from jax.experimental.pallas import tpu as pltpu
from jax.experimental.pallas import tpu_sc as plsc

N_NODES = 10000
D_NODE = 128
D_EDGE = 16
NOUT = 32
EHID = 128
STEPS = 6

NC, NS = 2, 16
NW = NC * NS
NP = 10240
NPW = NP // NW
EP = 163840
EPW = EP // NW
CE = 2560
BE = 2048
BN = 2048

_MESH = dict(core_axis_name="c", subcore_axis_name="s")


def _wid():
    return lax.axis_index("s") * NC + lax.axis_index("c")


def _permute_rows_body(ef_hbm, perm_hbm, out_hbm, idx_v, rows_v, sem):
    base_w = _wid() * EPW

    def chunk(c, _):
        base = base_w + c * CE
        pltpu.sync_copy(perm_hbm.at[pl.ds(base, CE)], idx_v)
        pltpu.async_copy(ef_hbm.at[idx_v], rows_v, sem).wait()
        pltpu.sync_copy(rows_v, out_hbm.at[pl.ds(base, CE)])
        return 0

    lax.fori_loop(0, EPW // CE, chunk, 0, unroll=False)


def _gather_rows_body(h_hbm, src_hbm, out_hbm, idx_v, rows_v, sem):
    base_w = _wid() * EPW

    def chunk(c, _):
        base = base_w + c * CE
        pltpu.sync_copy(src_hbm.at[pl.ds(base, CE)], idx_v)
        pltpu.async_copy(h_hbm.at[idx_v], rows_v, sem).wait()
        pltpu.sync_copy(rows_v, out_hbm.at[pl.ds(base, CE)])
        return 0

    lax.fori_loop(0, EPW // CE, chunk, 0, unroll=False)


def _pick(offs_v, target):
    ii = lax.iota(jnp.int32, 16)
    r = jnp.full((), -1, jnp.int32)
    for b in range(3):
        vb = offs_v[pl.ds(b * 16, 16)]
        r = jnp.maximum(
            r, jnp.max(jnp.where(ii + (b * 16) == target, vb, -1), axis=0))
    return r


def _scatter_body(mT_hbm, dst_hbm, offs_hbm, out_hbm, offs_v, dstC_v, mloc_v,
                  aggL_v, sem):
    w = _wid()
    pltpu.sync_copy(offs_hbm, offs_v)
    lo_node = w * NPW
    e_lo_raw = _pick(offs_v, w)
    e_hi = _pick(offs_v, w + 1)
    e_lo = (e_lo_raw // 16) * 16

    def zero(i, _):
        aggL_v[pl.ds(i * 16, 16)] = jnp.zeros((16,), jnp.float32)
        return 0

    lax.fori_loop(0, NPW * NOUT // 16, zero, 0, unroll=False)

    nch = (e_hi - e_lo + CE - 1) // CE

    def chunk(c, _):
        cstart = e_lo + c * CE
        base = pl.multiple_of(jnp.minimum(cstart, EP - CE), 16)
        pltpu.sync_copy(dst_hbm.at[pl.ds(base, CE)], dstC_v)
        pltpu.sync_copy(mT_hbm.at[:, pl.ds(base, CE)], mloc_v)

        def grp(g, _):
            gidx = base + g * 16 + lax.iota(jnp.int32, 16)
            dvec = dstC_v[pl.ds(g * 16, 16)]
            mask = ((dvec >= lo_node) & (dvec < lo_node + NPW)
                    & (gidx >= cstart))
            addr0 = (dvec - lo_node) * NOUT
            for o in range(NOUT):
                vals = mloc_v[o, pl.ds(g * 16, 16)]
                plsc.addupdate_scatter(aggL_v, [addr0 + o], vals, mask=mask)
            return 0

        lax.fori_loop(0, CE // 16, grp, 0, unroll=False)
        return 0

    lax.fori_loop(0, nch, chunk, 0, unroll=False)
    pltpu.sync_copy(aggL_v, out_hbm.at[pl.ds(w * NPW * NOUT, NPW * NOUT)])


def _score_body(uv_hbm, src_hbm, dst_hbm, out_hbm, uv_v, srcC_v, dstC_v,
                s_v, sem):
    base_w = _wid() * EPW
    pltpu.sync_copy(uv_hbm, uv_v)

    def chunk(c, _):
        base = base_w + c * CE
        pltpu.sync_copy(src_hbm.at[pl.ds(base, CE)], srcC_v)
        pltpu.sync_copy(dst_hbm.at[pl.ds(base, CE)], dstC_v)

        def grp(g, _):
            sl = pl.ds(g * 16, 16)
            zz = jnp.zeros((16,), jnp.int32)
            u16 = plsc.load_gather(uv_v, [srcC_v[sl], zz])
            v16 = plsc.load_gather(uv_v, [dstC_v[sl], zz + 1])
            s_v[sl] = u16 + v16
            return 0

        lax.fori_loop(0, CE // 16, grp, 0, unroll=False)
        pltpu.sync_copy(s_v, out_hbm.at[pl.ds(base, CE)])
        return 0

    lax.fori_loop(0, EPW // CE, chunk, 0, unroll=False)


def _sc_kernel(body, out_type, scratch_types):
    return pl.kernel(body, out_type=out_type,
                     mesh=plsc.VectorSubcoreMesh(**_MESH),
                     scratch_types=scratch_types,
                     compiler_params=pltpu.CompilerParams(
                         use_tc_tiling_on_sc=False,
                         needs_layout_passes=False))


def _mm(a, b):
    return jnp.matmul(a, b)


def _proj_body(nf_ref, w_ref, b_ref, out_ref):
    out_ref[...] = jnp.maximum(_mm(nf_ref[...], w_ref[...]) + b_ref[...], 0.0)


def _wesoa_body(ef_ref, w1_ref, b1_ref, w2_ref, b2_ref, out_ref):
    g = jnp.maximum(_mm(ef_ref[...], w1_ref[...]) + b1_ref[...], 0.0)
    out_ref[...] = lax.dot_general(
        w2_ref[...], g, (((0,), (1,)), ((), ())),
        preferred_element_type=jnp.float32) + b2_ref[...]


def _einsum_body(we_ref, hs_ref, out_ref):
    hT = jnp.transpose(hs_ref[...])
    terms = [we_ref[pl.ds(i * NOUT, NOUT), :] * hT[i:i + 1, :]
             for i in range(NOUT)]
    while len(terms) > 1:
        terms = [terms[j] + terms[j + 1] for j in range(0, len(terms), 2)]
    out_ref[...] = terms[0]


def _gru_body(agg_ref, hid_ref, bconv_ref, wih_ref, bih_ref, whh_ref,
              bhh_ref, out_ref):
    x = jnp.maximum(agg_ref[...] + bconv_ref[...], 0.0)
    hid = hid_ref[...]
    gi = _mm(x, wih_ref[...]) + bih_ref[...]
    gh = _mm(hid, whh_ref[...]) + bhh_ref[...]
    r = jax.nn.sigmoid(gi[:, :NOUT] + gh[:, :NOUT])
    z = jax.nn.sigmoid(gi[:, NOUT:2 * NOUT] + gh[:, NOUT:2 * NOUT])
    n = jnp.tanh(gi[:, 2 * NOUT:] + r * gh[:, 2 * NOUT:])
    out_ref[...] = (1.0 - z) * n + z * hid


def _uv_body(h_ref, wab_ref, badd_ref, out_ref):
    out_ref[...] = _mm(h_ref[...], wab_ref[...]) + badd_ref[...]


def _rep(shape, idx):
    return pl.BlockSpec(shape, idx)


def kernel(node_feats, edge_feats, edge_index, W_proj, b_proj, W_e1, b_e1,
           W_e2, b_e2, b_conv, W_ih, b_ih, W_hh, b_hh, W_pred, b_pred):
    f32 = jnp.float32
    src = edge_index[0]
    dst = edge_index[1]
    E = src.shape[0]

    perm = jnp.argsort(dst)
    dst_s = jnp.concatenate([dst[perm],
                             jnp.full((EP - E,), NP - 1, jnp.int32)])
    src_s = jnp.concatenate([src[perm], jnp.zeros((EP - E,), jnp.int32)])
    perm_p = jnp.concatenate([perm.astype(jnp.int32),
                              jnp.zeros((EP - E,), jnp.int32)])
    offs = jnp.searchsorted(dst_s, jnp.arange(33, dtype=jnp.int32) * NPW,
                            side="left").astype(jnp.int32)
    offs = jnp.concatenate([offs, jnp.zeros((31,), jnp.int32)])
    src_n = jnp.concatenate([src, jnp.zeros((EP - E,), jnp.int32)])
    dst_n = jnp.concatenate([dst, jnp.zeros((EP - E,), jnp.int32)])
    nf_p = jnp.pad(node_feats, ((0, NP - N_NODES), (0, 0)))

    ef_s = _sc_kernel(
        _permute_rows_body, jax.ShapeDtypeStruct((EP, D_EDGE), f32),
        [pltpu.VMEM((CE,), jnp.int32), pltpu.VMEM((CE, D_EDGE), f32),
         pltpu.SemaphoreType.DMA],
    )(edge_feats, perm_p)

    h0 = pl.pallas_call(
        _proj_body, grid=(NP // BN,),
        in_specs=[_rep((BN, D_NODE), lambda i: (i, 0)),
                  _rep((D_NODE, NOUT), lambda i: (0, 0)),
                  _rep((1, NOUT), lambda i: (0, 0))],
        out_specs=_rep((BN, NOUT), lambda i: (i, 0)),
        out_shape=jax.ShapeDtypeStruct((NP, NOUT), f32),
    )(nf_p, W_proj, b_proj.reshape(1, -1))

    we_soa = pl.pallas_call(
        _wesoa_body, grid=(EP // BE,),
        in_specs=[_rep((BE, D_EDGE), lambda i: (i, 0)),
                  _rep((D_EDGE, EHID), lambda i: (0, 0)),
                  _rep((1, EHID), lambda i: (0, 0)),
                  _rep((EHID, NOUT * NOUT), lambda i: (0, 0)),
                  _rep((NOUT * NOUT, 1), lambda i: (0, 0))],
        out_specs=_rep((NOUT * NOUT, BE), lambda i: (0, i)),
        out_shape=jax.ShapeDtypeStruct((NOUT * NOUT, EP), f32),
    )(ef_s, W_e1, b_e1.reshape(1, -1), W_e2, b_e2.reshape(-1, 1))

    gather_rows = _sc_kernel(
        _gather_rows_body, jax.ShapeDtypeStruct((EP, NOUT), f32),
        [pltpu.VMEM((CE,), jnp.int32), pltpu.VMEM((CE, NOUT), f32),
         pltpu.SemaphoreType.DMA],
    )
    scatter_agg = _sc_kernel(
        _scatter_body, jax.ShapeDtypeStruct((NP * NOUT,), f32),
        [pltpu.VMEM((64,), jnp.int32), pltpu.VMEM((CE,), jnp.int32),
         pltpu.VMEM((NOUT, CE), f32), pltpu.VMEM((NPW * NOUT,), f32),
         pltpu.SemaphoreType.DMA],
    )
    einsum_mT = pl.pallas_call(
        _einsum_body, grid=(EP // BE,),
        in_specs=[_rep((NOUT * NOUT, BE), lambda i: (0, i)),
                  _rep((BE, NOUT), lambda i: (i, 0))],
        out_specs=_rep((NOUT, BE), lambda i: (0, i)),
        out_shape=jax.ShapeDtypeStruct((NOUT, EP), f32),
    )
    gru = pl.pallas_call(
        _gru_body, grid=(NP // BN,),
        in_specs=[_rep((BN, NOUT), lambda i: (i, 0)),
                  _rep((BN, NOUT), lambda i: (i, 0)),
                  _rep((1, NOUT), lambda i: (0, 0)),
                  _rep((NOUT, 3 * NOUT), lambda i: (0, 0)),
                  _rep((1, 3 * NOUT), lambda i: (0, 0)),
                  _rep((NOUT, 3 * NOUT), lambda i: (0, 0)),
                  _rep((1, 3 * NOUT), lambda i: (0, 0))],
        out_specs=_rep((BN, NOUT), lambda i: (i, 0)),
        out_shape=jax.ShapeDtypeStruct((NP, NOUT), f32),
    )

    hid = h0
    for _ in range(STEPS):
        hs = gather_rows(hid, src_s)
        mT = einsum_mT(we_soa, hs)
        agg = scatter_agg(mT, dst_s, offs).reshape(NP, NOUT)
        hid = gru(agg, hid, b_conv.reshape(1, -1), W_ih,
                  b_ih.reshape(1, -1), W_hh, b_hh.reshape(1, -1))

    wab = jnp.concatenate([W_pred[:NOUT], W_pred[NOUT:]], axis=1)
    badd = jnp.stack([b_pred[0], jnp.zeros((), f32)]).reshape(1, 2)
    uv = pl.pallas_call(
        _uv_body, grid=(NP // BN,),
        in_specs=[_rep((BN, NOUT), lambda i: (i, 0)),
                  _rep((NOUT, 2), lambda i: (0, 0)),
                  _rep((1, 2), lambda i: (0, 0))],
        out_specs=_rep((BN, 2), lambda i: (i, 0)),
        out_shape=jax.ShapeDtypeStruct((NP, 2), f32),
    )(hid, wab, badd)

    score = _sc_kernel(
        _score_body, jax.ShapeDtypeStruct((EP,), f32),
        [pltpu.VMEM((NP, 2), f32), pltpu.VMEM((CE,), jnp.int32),
         pltpu.VMEM((CE,), jnp.int32), pltpu.VMEM((CE,), f32),
         pltpu.SemaphoreType.DMA],
    )(uv, src_n, dst_n)

    return score[:E].reshape(E, 1)
